# baseline (speedup 1.0000x reference)
import functools

import jax
import jax.numpy as jnp
from jax import lax
from jax.experimental import pallas as pl
from jax.experimental.pallas import tpu as pltpu
from jax.experimental.pallas import tpu_sc as plsc

_N = 10000
_NP = 10240
_E = 320000
_NC = 2
_NS = 16
_NW = _NC * _NS
_K = 128
_ER = _E // _K
_CPT = 78
_CPT2 = 156
_TAIL = _ER - _NW * _CPT
_RING = 2
_RPT = _NP // _NS


def _sc_mesh():
    return plsc.VectorSubcoreMesh(core_axis_name="c", subcore_axis_name="s")


_SC_PARAMS = pltpu.CompilerParams(use_tc_tiling_on_sc=False)


def _ring_pipeline(tab_sh, acc_sh, src_v, dst_v, buf_v, sem_g, sem_s, n):
    for b in range(_RING):
        pltpu.async_copy(tab_sh.at[src_v.at[b]], buf_v.at[b], sem_g[b])

    @pl.loop(0, n, step=_RING)
    def _(j):
        for b in range(_RING):
            i = j + b
            pltpu.make_async_copy(tab_sh.at[src_v.at[i]], buf_v.at[b],
                                  sem_g[b]).wait()
            pltpu.async_copy(buf_v.at[b], acc_sh.at[dst_v.at[i]], sem_s,
                             add=True)
        for b in range(_RING):
            i = j + b
            pltpu.make_async_copy(buf_v.at[b], acc_sh.at[dst_v.at[i]],
                                  sem_s).wait()

            @pl.when(i + _RING < n)
            def _():
                pltpu.async_copy(tab_sh.at[src_v.at[i + _RING]], buf_v.at[b],
                                 sem_g[b])


def _tail_sweep(w, tab_sh, acc_sh, tsrc_v, tdst_v, buf_v):
    @pl.when(w == 0)
    def _():
        for t in range(_TAIL):
            pltpu.sync_copy(tab_sh.at[tsrc_v.at[t]], buf_v.at[0])
            pltpu.sync_copy(buf_v.at[0], acc_sh.at[tdst_v.at[t]], add=True)


def _sc_degree(dst2d, ones, zeros16):
    @functools.partial(
        pl.kernel,
        out_type=jax.ShapeDtypeStruct((_NC, _NP, 16), jnp.float32),
        mesh=_sc_mesh(),
        compiler_params=_SC_PARAMS,
        scratch_types=[
            pltpu.VMEM((_CPT, _K), jnp.int32),
            pltpu.VMEM((_TAIL, _K), jnp.int32),
            pltpu.VMEM((_K, 16), jnp.float32),
            pltpu.VMEM_SHARED((_NP, 16), jnp.float32),
            pltpu.SemaphoreType.DMA,
        ],
    )
    def deg_kernel(dst_hbm, ones_hbm, zeros_hbm, out_hbm, dst_v, tdst_v,
                   ones_v, acc_sh, sem):
        c = lax.axis_index("c")
        s = lax.axis_index("s")
        w = c * _NS + s
        pltpu.sync_copy(dst_hbm.at[pl.ds(w * _CPT, _CPT)], dst_v)
        pltpu.sync_copy(ones_hbm, ones_v)

        @pl.when(w == 0)
        def _():
            pltpu.sync_copy(dst_hbm.at[pl.ds(_NW * _CPT, _TAIL)], tdst_v)

        pltpu.sync_copy(zeros_hbm.at[pl.ds(s * _RPT, _RPT)],
                        acc_sh.at[pl.ds(s * _RPT, _RPT)])
        plsc.subcore_barrier()

        @pl.loop(0, _CPT)
        def _(i):
            pltpu.async_copy(ones_v, acc_sh.at[dst_v.at[i]], sem, add=True)

        @pl.when(w == 0)
        def _():
            for t in range(_TAIL):
                pltpu.async_copy(ones_v, acc_sh.at[tdst_v.at[t]], sem, add=True)

        @pl.loop(0, _CPT)
        def _(i):
            pltpu.make_async_copy(ones_v, acc_sh.at[dst_v.at[i]], sem).wait()

        @pl.when(w == 0)
        def _():
            for t in range(_TAIL):
                pltpu.make_async_copy(ones_v, acc_sh.at[tdst_v.at[t]],
                                      sem).wait()

        plsc.subcore_barrier()
        pltpu.sync_copy(acc_sh.at[pl.ds(s * _RPT, _RPT)],
                        out_hbm.at[c, pl.ds(s * _RPT, _RPT)])

    return deg_kernel(dst2d, ones, zeros16)


def _sc_scatter(h):
    @functools.partial(
        pl.kernel,
        out_type=jax.ShapeDtypeStruct((_NC, _NP, h), jnp.float32),
        mesh=_sc_mesh(),
        compiler_params=_SC_PARAMS,
        scratch_types=[
            pltpu.VMEM((_CPT, _K), jnp.int32),
            pltpu.VMEM((_CPT, _K), jnp.int32),
            pltpu.VMEM((_TAIL, _K), jnp.int32),
            pltpu.VMEM((_TAIL, _K), jnp.int32),
            pltpu.VMEM((_RING, _K, h), jnp.float32),
            pltpu.VMEM_SHARED((_NP, h), jnp.float32),
            pltpu.VMEM_SHARED((_NP, h), jnp.float32),
            pltpu.SemaphoreType.DMA,
            pltpu.SemaphoreType.DMA,
            pltpu.SemaphoreType.DMA,
        ],
    )
    def scat_kernel(src_hbm, dst_hbm, g_hbm, zeros_hbm, out_hbm,
                    src_v, dst_v, tsrc_v, tdst_v, buf_v, acc_sh, tab_sh,
                    g0, g1, sem_s):
        c = lax.axis_index("c")
        s = lax.axis_index("s")
        w = c * _NS + s
        sem_g = (g0, g1)
        pltpu.sync_copy(src_hbm.at[pl.ds(w * _CPT, _CPT)], src_v)
        pltpu.sync_copy(dst_hbm.at[pl.ds(w * _CPT, _CPT)], dst_v)

        @pl.when(w == 0)
        def _():
            pltpu.sync_copy(src_hbm.at[pl.ds(_NW * _CPT, _TAIL)], tsrc_v)
            pltpu.sync_copy(dst_hbm.at[pl.ds(_NW * _CPT, _TAIL)], tdst_v)

        pltpu.sync_copy(g_hbm.at[pl.ds(s * _RPT, _RPT)],
                        tab_sh.at[pl.ds(s * _RPT, _RPT)])
        pltpu.sync_copy(zeros_hbm.at[pl.ds(s * _RPT, _RPT)],
                        acc_sh.at[pl.ds(s * _RPT, _RPT)])
        plsc.subcore_barrier()

        _ring_pipeline(tab_sh, acc_sh, src_v, dst_v, buf_v, sem_g, sem_s, _CPT)
        _tail_sweep(w, tab_sh, acc_sh, tsrc_v, tdst_v, buf_v)

        plsc.subcore_barrier()
        pltpu.sync_copy(acc_sh.at[pl.ds(s * _RPT, _RPT)],
                        out_hbm.at[c, pl.ds(s * _RPT, _RPT)])

    return scat_kernel


def _sc_scatter_wide():
    @functools.partial(
        pl.kernel,
        out_type=jax.ShapeDtypeStruct((_NC, _NP, 64), jnp.float32),
        mesh=_sc_mesh(),
        compiler_params=_SC_PARAMS,
        scratch_types=[
            pltpu.VMEM((_CPT, _K), jnp.int32),
            pltpu.VMEM((_CPT, _K), jnp.int32),
            pltpu.VMEM((_TAIL, _K), jnp.int32),
            pltpu.VMEM((_TAIL, _K), jnp.int32),
            pltpu.VMEM((_RING, _K, 64), jnp.float32),
            pltpu.VMEM_SHARED((_NP, 64), jnp.float32),
            pltpu.VMEM_SHARED((_NP, 64), jnp.float32),
            pltpu.SemaphoreType.DMA,
            pltpu.SemaphoreType.DMA,
            pltpu.SemaphoreType.DMA,
        ],
    )
    def scat_kernel(src_hbm, dst_hbm, glo_hbm, ghi_hbm, zeros_hbm, out_hbm,
                    src_v, dst_v, tsrc_v, tdst_v, buf_v, acc_sh, tab_sh,
                    g0, g1, sem_s):
        c = lax.axis_index("c")
        s = lax.axis_index("s")
        sem_g = (g0, g1)

        @pl.when(c == 0)
        def _():
            pltpu.sync_copy(glo_hbm.at[pl.ds(s * _RPT, _RPT)],
                            tab_sh.at[pl.ds(s * _RPT, _RPT)])

        @pl.when(c == 1)
        def _():
            pltpu.sync_copy(ghi_hbm.at[pl.ds(s * _RPT, _RPT)],
                            tab_sh.at[pl.ds(s * _RPT, _RPT)])

        pltpu.sync_copy(zeros_hbm.at[pl.ds(s * _RPT, _RPT)],
                        acc_sh.at[pl.ds(s * _RPT, _RPT)])

        @pl.when(s == 0)
        def _():
            pltpu.sync_copy(src_hbm.at[pl.ds(_NS * _CPT2, _TAIL)], tsrc_v)
            pltpu.sync_copy(dst_hbm.at[pl.ds(_NS * _CPT2, _TAIL)], tdst_v)

        plsc.subcore_barrier()

        for ph in range(2):
            pltpu.sync_copy(src_hbm.at[pl.ds(s * _CPT2 + ph * _CPT, _CPT)],
                            src_v)
            pltpu.sync_copy(dst_hbm.at[pl.ds(s * _CPT2 + ph * _CPT, _CPT)],
                            dst_v)
            _ring_pipeline(tab_sh, acc_sh, src_v, dst_v, buf_v, sem_g, sem_s,
                           _CPT)

        _tail_sweep(s, tab_sh, acc_sh, tsrc_v, tdst_v, buf_v)

        plsc.subcore_barrier()
        pltpu.sync_copy(acc_sh.at[pl.ds(s * _RPT, _RPT)],
                        out_hbm.at[c, pl.ds(s * _RPT, _RPT)])

    return scat_kernel


def _tc_h1(x, w1):
    def body(x_ref, w_ref, o_ref):
        o_ref[0:_N, :] = jnp.dot(x_ref[...], w_ref[...],
                                 preferred_element_type=jnp.float32)
        o_ref[_N:_NP, :] = jnp.zeros((_NP - _N, w1.shape[1]), jnp.float32)

    return pl.pallas_call(
        body,
        out_shape=jax.ShapeDtypeStruct((_NP, w1.shape[1]), jnp.float32),
    )(x, w1)


def _tc_scale(deg16, h1):
    def body(d_ref, h_ref, dinv_ref, glo_ref, ghi_ref):
        deg = d_ref[0, :, 0:1] + d_ref[1, :, 0:1] + 1.0
        dinv = lax.rsqrt(deg)
        rows = lax.broadcasted_iota(jnp.int32, (_NP, 1), 0)
        dinv = jnp.where(rows < _N, dinv, 0.0)
        dinv_ref[...] = dinv
        glo_ref[...] = dinv * h_ref[:, 0:64]
        ghi_ref[...] = dinv * h_ref[:, 64:128]

    return pl.pallas_call(
        body,
        out_shape=[
            jax.ShapeDtypeStruct((_NP, 1), jnp.float32),
            jax.ShapeDtypeStruct((_NP, 64), jnp.float32),
            jax.ShapeDtypeStruct((_NP, 64), jnp.float32),
        ],
    )(deg16, h1)


def _tc_mid1(s_cols, glo, ghi, dinv, b, wn):
    def body(s_ref, glo_ref, ghi_ref, dinv_ref, b_ref, w_ref, o_ref):
        dv = dinv_ref[...]
        t = jnp.concatenate(
            [s_ref[0] + glo_ref[...], s_ref[1] + ghi_ref[...]], axis=1)
        t = jnp.maximum(dv * t + b_ref[...], 0.0)
        o_ref[...] = dv * jnp.dot(t, w_ref[...],
                                  preferred_element_type=jnp.float32)

    return pl.pallas_call(
        body,
        out_shape=jax.ShapeDtypeStruct((_NP, wn.shape[1]), jnp.float32),
    )(s_cols, glo, ghi, dinv, b, wn)


def _tc_mid(s_parts, g, dinv, b, wn):
    def body(s_ref, g_ref, dinv_ref, b_ref, w_ref, o_ref):
        dv = dinv_ref[...]
        t = jnp.maximum(dv * (s_ref[0] + s_ref[1] + g_ref[...]) + b_ref[...], 0.0)
        o_ref[...] = dv * jnp.dot(t, w_ref[...],
                                  preferred_element_type=jnp.float32)

    return pl.pallas_call(
        body,
        out_shape=jax.ShapeDtypeStruct((_NP, wn.shape[1]), jnp.float32),
    )(s_parts, g, dinv, b, wn)


def _tc_post(s_parts, g, dinv, b):
    def body(s_ref, g_ref, dinv_ref, b_ref, o_ref):
        o_ref[...] = jnp.maximum(
            dinv_ref[0:_N] * (s_ref[0, 0:_N, 0:8] + s_ref[1, 0:_N, 0:8]
                              + g_ref[0:_N, 0:8]) + b_ref[:, 0:8], 0.0)

    return pl.pallas_call(
        body,
        out_shape=jax.ShapeDtypeStruct((_N, 8), jnp.float32),
    )(s_parts, g, dinv, b)


def _scatter_any(src2d, dst2d, g, h):
    zeros = jnp.zeros((_NP, h), jnp.float32)
    return _sc_scatter(h)(src2d, dst2d, g, zeros)


def kernel(x, edge_index, W1, b1, W2, b2, W3, b3, W4, b4, W5, b5):
    src2d = edge_index[0].reshape(_ER, _K)
    dst2d = edge_index[1].reshape(_ER, _K)

    w5p = jnp.pad(W5, ((0, 0), (0, 8)))
    b5p = jnp.pad(b5, (0, 8))

    ones = jnp.ones((_K, 16), jnp.float32)
    z16 = jnp.zeros((_NP, 16), jnp.float32)

    deg16 = _sc_degree(dst2d, ones, z16)
    h1 = _tc_h1(x, W1)
    dinv, glo, ghi = _tc_scale(deg16, h1)

    z64 = jnp.zeros((_NP, 64), jnp.float32)
    s_cols = _sc_scatter_wide()(src2d, dst2d, glo, ghi, z64)
    g = _tc_mid1(s_cols, glo, ghi, dinv, b1.reshape(1, -1), W2)

    layer_w = [(64, b2, W3), (32, b3, W4), (16, b4, w5p)]
    for h, b, wn in layer_w:
        s_parts = _scatter_any(src2d, dst2d, g, h)
        g = _tc_mid(s_parts, g, dinv, b.reshape(1, -1), wn)

    s_parts = _scatter_any(src2d, dst2d, g, 16)
    return _tc_post(s_parts, g, dinv, b5p.reshape(1, -1))


# speedup vs baseline: 1.0104x; 1.0104x over previous
import functools

import jax
import jax.numpy as jnp
from jax import lax
from jax.experimental import pallas as pl
from jax.experimental.pallas import tpu as pltpu
from jax.experimental.pallas import tpu_sc as plsc

_N = 10000
_NP = 10240
_E = 320000
_NC = 2
_NS = 16
_NW = _NC * _NS
_K = 128
_ER = _E // _K
_CPT = 78
_CPT2 = 156
_TAIL = _ER - _NW * _CPT
_RING = 2
_RING3 = 3
_RPT = _NP // _NS


def _sc_mesh():
    return plsc.VectorSubcoreMesh(core_axis_name="c", subcore_axis_name="s")


_SC_PARAMS = pltpu.CompilerParams(use_tc_tiling_on_sc=False)


def _ring_pipeline(tab_sh, acc_sh, src_v, dst_v, buf_v, sem_g, sem_s, n,
                   ring):
    for b in range(ring):
        pltpu.async_copy(tab_sh.at[src_v.at[b]], buf_v.at[b], sem_g[b])

    @pl.loop(0, n, step=ring)
    def _(j):
        for b in range(ring):
            i = j + b
            pltpu.make_async_copy(tab_sh.at[src_v.at[i]], buf_v.at[b],
                                  sem_g[b]).wait()
            pltpu.async_copy(buf_v.at[b], acc_sh.at[dst_v.at[i]], sem_s,
                             add=True)
        for b in range(ring):
            i = j + b
            pltpu.make_async_copy(buf_v.at[b], acc_sh.at[dst_v.at[i]],
                                  sem_s).wait()

            @pl.when(i + ring < n)
            def _():
                pltpu.async_copy(tab_sh.at[src_v.at[i + ring]], buf_v.at[b],
                                 sem_g[b])


def _tail_sweep(w, tab_sh, acc_sh, tsrc_v, tdst_v, buf_v):
    @pl.when(w == 0)
    def _():
        for t in range(_TAIL):
            pltpu.sync_copy(tab_sh.at[tsrc_v.at[t]], buf_v.at[0])
            pltpu.sync_copy(buf_v.at[0], acc_sh.at[tdst_v.at[t]], add=True)


def _sc_degree(dst2d, ones, zeros16):
    @functools.partial(
        pl.kernel,
        out_type=jax.ShapeDtypeStruct((_NC, _NP, 16), jnp.float32),
        mesh=_sc_mesh(),
        compiler_params=_SC_PARAMS,
        scratch_types=[
            pltpu.VMEM((_CPT, _K), jnp.int32),
            pltpu.VMEM((_TAIL, _K), jnp.int32),
            pltpu.VMEM((_K, 16), jnp.float32),
            pltpu.VMEM_SHARED((_NP, 16), jnp.float32),
            pltpu.SemaphoreType.DMA,
        ],
    )
    def deg_kernel(dst_hbm, ones_hbm, zeros_hbm, out_hbm, dst_v, tdst_v,
                   ones_v, acc_sh, sem):
        c = lax.axis_index("c")
        s = lax.axis_index("s")
        w = c * _NS + s
        pltpu.sync_copy(dst_hbm.at[pl.ds(w * _CPT, _CPT)], dst_v)
        pltpu.sync_copy(ones_hbm, ones_v)

        @pl.when(w == 0)
        def _():
            pltpu.sync_copy(dst_hbm.at[pl.ds(_NW * _CPT, _TAIL)], tdst_v)

        pltpu.sync_copy(zeros_hbm.at[pl.ds(s * _RPT, _RPT)],
                        acc_sh.at[pl.ds(s * _RPT, _RPT)])
        plsc.subcore_barrier()

        @pl.loop(0, _CPT)
        def _(i):
            pltpu.async_copy(ones_v, acc_sh.at[dst_v.at[i]], sem, add=True)

        @pl.when(w == 0)
        def _():
            for t in range(_TAIL):
                pltpu.async_copy(ones_v, acc_sh.at[tdst_v.at[t]], sem, add=True)

        @pl.loop(0, _CPT)
        def _(i):
            pltpu.make_async_copy(ones_v, acc_sh.at[dst_v.at[i]], sem).wait()

        @pl.when(w == 0)
        def _():
            for t in range(_TAIL):
                pltpu.make_async_copy(ones_v, acc_sh.at[tdst_v.at[t]],
                                      sem).wait()

        plsc.subcore_barrier()
        pltpu.sync_copy(acc_sh.at[pl.ds(s * _RPT, _RPT)],
                        out_hbm.at[c, pl.ds(s * _RPT, _RPT)])

    return deg_kernel(dst2d, ones, zeros16)


def _sc_scatter(h):
    ring = _RING if h > 32 else _RING3

    @functools.partial(
        pl.kernel,
        out_type=jax.ShapeDtypeStruct((_NC, _NP, h), jnp.float32),
        mesh=_sc_mesh(),
        compiler_params=_SC_PARAMS,
        scratch_types=[
            pltpu.VMEM((_CPT, _K), jnp.int32),
            pltpu.VMEM((_CPT, _K), jnp.int32),
            pltpu.VMEM((_TAIL, _K), jnp.int32),
            pltpu.VMEM((_TAIL, _K), jnp.int32),
            pltpu.VMEM((ring, _K, h), jnp.float32),
            pltpu.VMEM_SHARED((_NP, h), jnp.float32),
            pltpu.VMEM_SHARED((_NP, h), jnp.float32),
        ] + [pltpu.SemaphoreType.DMA] * (ring + 1),
    )
    def scat_kernel(src_hbm, dst_hbm, g_hbm, zeros_hbm, out_hbm,
                    src_v, dst_v, tsrc_v, tdst_v, buf_v, acc_sh, tab_sh,
                    *sems):
        c = lax.axis_index("c")
        s = lax.axis_index("s")
        w = c * _NS + s
        sem_g, sem_s = sems[:ring], sems[ring]
        pltpu.sync_copy(src_hbm.at[pl.ds(w * _CPT, _CPT)], src_v)
        pltpu.sync_copy(dst_hbm.at[pl.ds(w * _CPT, _CPT)], dst_v)

        @pl.when(w == 0)
        def _():
            pltpu.sync_copy(src_hbm.at[pl.ds(_NW * _CPT, _TAIL)], tsrc_v)
            pltpu.sync_copy(dst_hbm.at[pl.ds(_NW * _CPT, _TAIL)], tdst_v)

        pltpu.sync_copy(g_hbm.at[pl.ds(s * _RPT, _RPT)],
                        tab_sh.at[pl.ds(s * _RPT, _RPT)])
        pltpu.sync_copy(zeros_hbm.at[pl.ds(s * _RPT, _RPT)],
                        acc_sh.at[pl.ds(s * _RPT, _RPT)])
        plsc.subcore_barrier()

        _ring_pipeline(tab_sh, acc_sh, src_v, dst_v, buf_v, sem_g, sem_s,
                       _CPT, ring)
        _tail_sweep(w, tab_sh, acc_sh, tsrc_v, tdst_v, buf_v)

        plsc.subcore_barrier()
        pltpu.sync_copy(acc_sh.at[pl.ds(s * _RPT, _RPT)],
                        out_hbm.at[c, pl.ds(s * _RPT, _RPT)])

    return scat_kernel


def _sc_scatter_wide():
    @functools.partial(
        pl.kernel,
        out_type=jax.ShapeDtypeStruct((_NC, _NP, 64), jnp.float32),
        mesh=_sc_mesh(),
        compiler_params=_SC_PARAMS,
        scratch_types=[
            pltpu.VMEM((_CPT, _K), jnp.int32),
            pltpu.VMEM((_CPT, _K), jnp.int32),
            pltpu.VMEM((_TAIL, _K), jnp.int32),
            pltpu.VMEM((_TAIL, _K), jnp.int32),
            pltpu.VMEM((_RING, _K, 64), jnp.float32),
            pltpu.VMEM_SHARED((_NP, 64), jnp.float32),
            pltpu.VMEM_SHARED((_NP, 64), jnp.float32),
            pltpu.SemaphoreType.DMA,
            pltpu.SemaphoreType.DMA,
            pltpu.SemaphoreType.DMA,
        ],
    )
    def scat_kernel(src_hbm, dst_hbm, glo_hbm, ghi_hbm, zeros_hbm, out_hbm,
                    src_v, dst_v, tsrc_v, tdst_v, buf_v, acc_sh, tab_sh,
                    g0, g1, sem_s):
        c = lax.axis_index("c")
        s = lax.axis_index("s")
        sem_g = (g0, g1)

        @pl.when(c == 0)
        def _():
            pltpu.sync_copy(glo_hbm.at[pl.ds(s * _RPT, _RPT)],
                            tab_sh.at[pl.ds(s * _RPT, _RPT)])

        @pl.when(c == 1)
        def _():
            pltpu.sync_copy(ghi_hbm.at[pl.ds(s * _RPT, _RPT)],
                            tab_sh.at[pl.ds(s * _RPT, _RPT)])

        pltpu.sync_copy(zeros_hbm.at[pl.ds(s * _RPT, _RPT)],
                        acc_sh.at[pl.ds(s * _RPT, _RPT)])

        @pl.when(s == 0)
        def _():
            pltpu.sync_copy(src_hbm.at[pl.ds(_NS * _CPT2, _TAIL)], tsrc_v)
            pltpu.sync_copy(dst_hbm.at[pl.ds(_NS * _CPT2, _TAIL)], tdst_v)

        plsc.subcore_barrier()

        for ph in range(2):
            pltpu.sync_copy(src_hbm.at[pl.ds(s * _CPT2 + ph * _CPT, _CPT)],
                            src_v)
            pltpu.sync_copy(dst_hbm.at[pl.ds(s * _CPT2 + ph * _CPT, _CPT)],
                            dst_v)
            _ring_pipeline(tab_sh, acc_sh, src_v, dst_v, buf_v, sem_g, sem_s,
                           _CPT, _RING)

        _tail_sweep(s, tab_sh, acc_sh, tsrc_v, tdst_v, buf_v)

        plsc.subcore_barrier()
        pltpu.sync_copy(acc_sh.at[pl.ds(s * _RPT, _RPT)],
                        out_hbm.at[c, pl.ds(s * _RPT, _RPT)])

    return scat_kernel


def _tc_h1(x, w1):
    def body(x_ref, w_ref, o_ref):
        o_ref[0:_N, :] = jnp.dot(x_ref[...], w_ref[...],
                                 preferred_element_type=jnp.float32)
        o_ref[_N:_NP, :] = jnp.zeros((_NP - _N, w1.shape[1]), jnp.float32)

    return pl.pallas_call(
        body,
        out_shape=jax.ShapeDtypeStruct((_NP, w1.shape[1]), jnp.float32),
    )(x, w1)


def _tc_scale(deg16, h1):
    def body(d_ref, h_ref, dinv_ref, glo_ref, ghi_ref):
        deg = d_ref[0, :, 0:1] + d_ref[1, :, 0:1] + 1.0
        dinv = lax.rsqrt(deg)
        rows = lax.broadcasted_iota(jnp.int32, (_NP, 1), 0)
        dinv = jnp.where(rows < _N, dinv, 0.0)
        dinv_ref[...] = dinv
        glo_ref[...] = dinv * h_ref[:, 0:64]
        ghi_ref[...] = dinv * h_ref[:, 64:128]

    return pl.pallas_call(
        body,
        out_shape=[
            jax.ShapeDtypeStruct((_NP, 1), jnp.float32),
            jax.ShapeDtypeStruct((_NP, 64), jnp.float32),
            jax.ShapeDtypeStruct((_NP, 64), jnp.float32),
        ],
    )(deg16, h1)


def _tc_mid1(s_cols, glo, ghi, dinv, b, wn):
    def body(s_ref, glo_ref, ghi_ref, dinv_ref, b_ref, w_ref, o_ref):
        dv = dinv_ref[...]
        t = jnp.concatenate(
            [s_ref[0] + glo_ref[...], s_ref[1] + ghi_ref[...]], axis=1)
        t = jnp.maximum(dv * t + b_ref[...], 0.0)
        o_ref[...] = dv * jnp.dot(t, w_ref[...],
                                  preferred_element_type=jnp.float32)

    return pl.pallas_call(
        body,
        out_shape=jax.ShapeDtypeStruct((_NP, wn.shape[1]), jnp.float32),
    )(s_cols, glo, ghi, dinv, b, wn)


def _tc_mid(s_parts, g, dinv, b, wn):
    def body(s_ref, g_ref, dinv_ref, b_ref, w_ref, o_ref):
        dv = dinv_ref[...]
        t = jnp.maximum(dv * (s_ref[0] + s_ref[1] + g_ref[...]) + b_ref[...], 0.0)
        o_ref[...] = dv * jnp.dot(t, w_ref[...],
                                  preferred_element_type=jnp.float32)

    return pl.pallas_call(
        body,
        out_shape=jax.ShapeDtypeStruct((_NP, wn.shape[1]), jnp.float32),
    )(s_parts, g, dinv, b, wn)


def _tc_post(s_parts, g, dinv, b):
    def body(s_ref, g_ref, dinv_ref, b_ref, o_ref):
        o_ref[...] = jnp.maximum(
            dinv_ref[0:_N] * (s_ref[0, 0:_N, 0:8] + s_ref[1, 0:_N, 0:8]
                              + g_ref[0:_N, 0:8]) + b_ref[:, 0:8], 0.0)

    return pl.pallas_call(
        body,
        out_shape=jax.ShapeDtypeStruct((_N, 8), jnp.float32),
    )(s_parts, g, dinv, b)


def _scatter_any(src2d, dst2d, g, h):
    zeros = jnp.zeros((_NP, h), jnp.float32)
    return _sc_scatter(h)(src2d, dst2d, g, zeros)


def kernel(x, edge_index, W1, b1, W2, b2, W3, b3, W4, b4, W5, b5):
    src2d = edge_index[0].reshape(_ER, _K)
    dst2d = edge_index[1].reshape(_ER, _K)

    w5p = jnp.pad(W5, ((0, 0), (0, 8)))
    b5p = jnp.pad(b5, (0, 8))

    ones = jnp.ones((_K, 16), jnp.float32)
    z16 = jnp.zeros((_NP, 16), jnp.float32)

    deg16 = _sc_degree(dst2d, ones, z16)
    h1 = _tc_h1(x, W1)
    dinv, glo, ghi = _tc_scale(deg16, h1)

    z64 = jnp.zeros((_NP, 64), jnp.float32)
    s_cols = _sc_scatter_wide()(src2d, dst2d, glo, ghi, z64)
    g = _tc_mid1(s_cols, glo, ghi, dinv, b1.reshape(1, -1), W2)

    layer_w = [(64, b2, W3), (32, b3, W4), (16, b4, w5p)]
    for h, b, wn in layer_w:
        s_parts = _scatter_any(src2d, dst2d, g, h)
        g = _tc_mid(s_parts, g, dinv, b.reshape(1, -1), wn)

    s_parts = _scatter_any(src2d, dst2d, g, 16)
    return _tc_post(s_parts, g, dinv, b5p.reshape(1, -1))


# speedup vs baseline: 1.0295x; 1.0189x over previous
import functools

import jax
import jax.numpy as jnp
from jax import lax
from jax.experimental import pallas as pl
from jax.experimental.pallas import tpu as pltpu
from jax.experimental.pallas import tpu_sc as plsc

_N = 10000
_NP = 10240
_E = 320000
_NC = 2
_NS = 16
_NW = _NC * _NS
_K = 125
_NCH = 80
_NCH2 = 160
_RING = 2
_RING4 = 4
_RPT = _NP // _NS


def _sc_mesh():
    return plsc.VectorSubcoreMesh(core_axis_name="c", subcore_axis_name="s")


_SC_PARAMS = pltpu.CompilerParams(use_tc_tiling_on_sc=False)


def _sc_degree(dst3, ones, zeros16):
    @functools.partial(
        pl.kernel,
        out_type=jax.ShapeDtypeStruct((_NC, _NP, 16), jnp.float32),
        mesh=_sc_mesh(),
        compiler_params=_SC_PARAMS,
        scratch_types=[
            pltpu.VMEM((_NCH, _K), jnp.int32),
            pltpu.VMEM((_K, 16), jnp.float32),
            pltpu.VMEM_SHARED((_NP, 16), jnp.float32),
            pltpu.SemaphoreType.DMA,
        ],
    )
    def deg_kernel(dst_hbm, ones_hbm, zeros_hbm, out_hbm, dst_v, ones_v, acc_sh,
                   sem):
        c = lax.axis_index("c")
        s = lax.axis_index("s")
        w = c * _NS + s
        pltpu.sync_copy(dst_hbm.at[w], dst_v)
        pltpu.sync_copy(ones_hbm, ones_v)
        pltpu.sync_copy(zeros_hbm.at[pl.ds(s * _RPT, _RPT)],
                        acc_sh.at[pl.ds(s * _RPT, _RPT)])
        plsc.subcore_barrier()

        @pl.loop(0, _NCH)
        def _(i):
            pltpu.async_copy(ones_v, acc_sh.at[dst_v.at[i]], sem, add=True)

        @pl.loop(0, _NCH)
        def _(i):
            pltpu.make_async_copy(ones_v, acc_sh.at[dst_v.at[i]], sem).wait()

        plsc.subcore_barrier()
        pltpu.sync_copy(acc_sh.at[pl.ds(s * _RPT, _RPT)],
                        out_hbm.at[c, pl.ds(s * _RPT, _RPT)])

    return deg_kernel(dst3, ones, zeros16)


def _sc_scatter(h):
    ring = _RING if h > 32 else _RING4

    @functools.partial(
        pl.kernel,
        out_type=jax.ShapeDtypeStruct((_NC, _NP, h), jnp.float32),
        mesh=_sc_mesh(),
        compiler_params=_SC_PARAMS,
        scratch_types=[
            pltpu.VMEM((_NCH, _K), jnp.int32),
            pltpu.VMEM((_NCH, _K), jnp.int32),
            pltpu.VMEM((ring, _K, h), jnp.float32),
            pltpu.VMEM_SHARED((_NP, h), jnp.float32),
            pltpu.VMEM_SHARED((_NP, h), jnp.float32),
        ] + [pltpu.SemaphoreType.DMA] * (ring + 1),
    )
    def scat_kernel(src_hbm, dst_hbm, g_hbm, zeros_hbm, out_hbm,
                    src_v, dst_v, buf_v, acc_sh, tab_sh, *sems):
        c = lax.axis_index("c")
        s = lax.axis_index("s")
        w = c * _NS + s
        sem_g, sem_s = sems[:ring], sems[ring]
        pltpu.sync_copy(src_hbm.at[w], src_v)
        pltpu.sync_copy(dst_hbm.at[w], dst_v)
        pltpu.sync_copy(g_hbm.at[pl.ds(s * _RPT, _RPT)],
                        tab_sh.at[pl.ds(s * _RPT, _RPT)])
        pltpu.sync_copy(zeros_hbm.at[pl.ds(s * _RPT, _RPT)],
                        acc_sh.at[pl.ds(s * _RPT, _RPT)])
        plsc.subcore_barrier()

        for b in range(ring):
            pltpu.async_copy(tab_sh.at[src_v.at[b]], buf_v.at[b], sem_g[b])

        @pl.loop(0, _NCH, step=ring)
        def _(j):
            for b in range(ring):
                i = j + b
                pltpu.make_async_copy(tab_sh.at[src_v.at[i]], buf_v.at[b],
                                      sem_g[b]).wait()
                pltpu.async_copy(buf_v.at[b], acc_sh.at[dst_v.at[i]], sem_s,
                                 add=True)
            for b in range(ring):
                i = j + b
                pltpu.make_async_copy(buf_v.at[b], acc_sh.at[dst_v.at[i]],
                                      sem_s).wait()

                @pl.when(i + ring < _NCH)
                def _():
                    pltpu.async_copy(tab_sh.at[src_v.at[i + ring]], buf_v.at[b],
                                     sem_g[b])

        plsc.subcore_barrier()
        pltpu.sync_copy(acc_sh.at[pl.ds(s * _RPT, _RPT)],
                        out_hbm.at[c, pl.ds(s * _RPT, _RPT)])

    return scat_kernel


def _sc_scatter_wide():
    @functools.partial(
        pl.kernel,
        out_type=jax.ShapeDtypeStruct((_NC, _NP, 64), jnp.float32),
        mesh=_sc_mesh(),
        compiler_params=_SC_PARAMS,
        scratch_types=[
            pltpu.VMEM((_NCH, _K), jnp.int32),
            pltpu.VMEM((_NCH, _K), jnp.int32),
            pltpu.VMEM((_RING, _K, 64), jnp.float32),
            pltpu.VMEM_SHARED((_NP, 64), jnp.float32),
            pltpu.VMEM_SHARED((_NP, 64), jnp.float32),
            pltpu.SemaphoreType.DMA,
            pltpu.SemaphoreType.DMA,
            pltpu.SemaphoreType.DMA,
        ],
    )
    def scat_kernel(src_hbm, dst_hbm, glo_hbm, ghi_hbm, zeros_hbm, out_hbm,
                    src_v, dst_v, buf_v, acc_sh, tab_sh, g0, g1, sem_s):
        c = lax.axis_index("c")
        s = lax.axis_index("s")
        sem_g = (g0, g1)

        @pl.when(c == 0)
        def _():
            pltpu.sync_copy(glo_hbm.at[pl.ds(s * _RPT, _RPT)],
                            tab_sh.at[pl.ds(s * _RPT, _RPT)])

        @pl.when(c == 1)
        def _():
            pltpu.sync_copy(ghi_hbm.at[pl.ds(s * _RPT, _RPT)],
                            tab_sh.at[pl.ds(s * _RPT, _RPT)])

        pltpu.sync_copy(zeros_hbm.at[pl.ds(s * _RPT, _RPT)],
                        acc_sh.at[pl.ds(s * _RPT, _RPT)])
        plsc.subcore_barrier()

        for ph in range(2):
            pltpu.sync_copy(src_hbm.at[s, pl.ds(ph * _NCH, _NCH)], src_v)
            pltpu.sync_copy(dst_hbm.at[s, pl.ds(ph * _NCH, _NCH)], dst_v)

            for b in range(_RING):
                pltpu.async_copy(tab_sh.at[src_v.at[b]], buf_v.at[b], sem_g[b])

            @pl.loop(0, _NCH, step=_RING)
            def _(j):
                for b in range(_RING):
                    i = j + b
                    pltpu.make_async_copy(tab_sh.at[src_v.at[i]], buf_v.at[b],
                                          sem_g[b]).wait()
                    pltpu.async_copy(buf_v.at[b], acc_sh.at[dst_v.at[i]], sem_s,
                                     add=True)
                for b in range(_RING):
                    i = j + b
                    pltpu.make_async_copy(buf_v.at[b], acc_sh.at[dst_v.at[i]],
                                          sem_s).wait()

                    @pl.when(i + _RING < _NCH)
                    def _():
                        pltpu.async_copy(tab_sh.at[src_v.at[i + _RING]],
                                         buf_v.at[b], sem_g[b])

        plsc.subcore_barrier()
        pltpu.sync_copy(acc_sh.at[pl.ds(s * _RPT, _RPT)],
                        out_hbm.at[c, pl.ds(s * _RPT, _RPT)])

    return scat_kernel


def _tc_h1(x, w1):
    def body(x_ref, w_ref, o_ref):
        o_ref[0:_N, :] = jnp.dot(x_ref[...], w_ref[...],
                                 preferred_element_type=jnp.float32)
        o_ref[_N:_NP, :] = jnp.zeros((_NP - _N, w1.shape[1]), jnp.float32)

    return pl.pallas_call(
        body,
        out_shape=jax.ShapeDtypeStruct((_NP, w1.shape[1]), jnp.float32),
    )(x, w1)


def _tc_scale(deg16, h1):
    def body(d_ref, h_ref, dinv_ref, glo_ref, ghi_ref):
        deg = d_ref[0, :, 0:1] + d_ref[1, :, 0:1] + 1.0
        dinv = lax.rsqrt(deg)
        rows = lax.broadcasted_iota(jnp.int32, (_NP, 1), 0)
        dinv = jnp.where(rows < _N, dinv, 0.0)
        dinv_ref[...] = dinv
        glo_ref[...] = dinv * h_ref[:, 0:64]
        ghi_ref[...] = dinv * h_ref[:, 64:128]

    return pl.pallas_call(
        body,
        out_shape=[
            jax.ShapeDtypeStruct((_NP, 1), jnp.float32),
            jax.ShapeDtypeStruct((_NP, 64), jnp.float32),
            jax.ShapeDtypeStruct((_NP, 64), jnp.float32),
        ],
    )(deg16, h1)


def _tc_mid1(s_cols, glo, ghi, dinv, b, wn):
    def body(s_ref, glo_ref, ghi_ref, dinv_ref, b_ref, w_ref, o_ref):
        dv = dinv_ref[...]
        t = jnp.concatenate(
            [s_ref[0] + glo_ref[...], s_ref[1] + ghi_ref[...]], axis=1)
        t = jnp.maximum(dv * t + b_ref[...], 0.0)
        o_ref[...] = dv * jnp.dot(t, w_ref[...],
                                  preferred_element_type=jnp.float32)

    return pl.pallas_call(
        body,
        out_shape=jax.ShapeDtypeStruct((_NP, wn.shape[1]), jnp.float32),
    )(s_cols, glo, ghi, dinv, b, wn)


def _tc_mid(s_parts, g, dinv, b, wn):
    def body(s_ref, g_ref, dinv_ref, b_ref, w_ref, o_ref):
        dv = dinv_ref[...]
        t = jnp.maximum(dv * (s_ref[0] + s_ref[1] + g_ref[...]) + b_ref[...], 0.0)
        o_ref[...] = dv * jnp.dot(t, w_ref[...],
                                  preferred_element_type=jnp.float32)

    return pl.pallas_call(
        body,
        out_shape=jax.ShapeDtypeStruct((_NP, wn.shape[1]), jnp.float32),
    )(s_parts, g, dinv, b, wn)


def _tc_post(s_parts, g, dinv, b):
    def body(s_ref, g_ref, dinv_ref, b_ref, o_ref):
        o_ref[...] = jnp.maximum(
            dinv_ref[0:_N] * (s_ref[0, 0:_N, 0:8] + s_ref[1, 0:_N, 0:8]
                              + g_ref[0:_N, 0:8]) + b_ref[:, 0:8], 0.0)

    return pl.pallas_call(
        body,
        out_shape=jax.ShapeDtypeStruct((_N, 8), jnp.float32),
    )(s_parts, g, dinv, b)


def _scatter_any(src3, dst3, g, h):
    zeros = jnp.zeros((_NP, h), jnp.float32)
    return _sc_scatter(h)(src3, dst3, g, zeros)


def kernel(x, edge_index, W1, b1, W2, b2, W3, b3, W4, b4, W5, b5):
    src3 = edge_index[0].reshape(_NW, _NCH, _K)
    dst3 = edge_index[1].reshape(_NW, _NCH, _K)
    src2 = edge_index[0].reshape(_NS, _NCH2, _K)
    dst2 = edge_index[1].reshape(_NS, _NCH2, _K)

    w5p = jnp.pad(W5, ((0, 0), (0, 8)))
    b5p = jnp.pad(b5, (0, 8))

    ones = jnp.ones((_K, 16), jnp.float32)
    z16 = jnp.zeros((_NP, 16), jnp.float32)

    deg16 = _sc_degree(dst3, ones, z16)
    h1 = _tc_h1(x, W1)
    dinv, glo, ghi = _tc_scale(deg16, h1)

    z64 = jnp.zeros((_NP, 64), jnp.float32)
    s_cols = _sc_scatter_wide()(src2, dst2, glo, ghi, z64)
    g = _tc_mid1(s_cols, glo, ghi, dinv, b1.reshape(1, -1), W2)

    layer_w = [(64, b2, W3), (32, b3, W4), (16, b4, w5p)]
    for h, b, wn in layer_w:
        s_parts = _scatter_any(src3, dst3, g, h)
        g = _tc_mid(s_parts, g, dinv, b.reshape(1, -1), wn)

    s_parts = _scatter_any(src3, dst3, g, 16)
    return _tc_post(s_parts, g, dinv, b5p.reshape(1, -1))
